# Initial kernel scaffold; baseline (speedup 1.0000x reference)
import jax
import jax.numpy as jnp
from jax.experimental import pallas as pl


def kernel(x, edge_index, weight, w_ih, w_hh, b_ih, b_hh):
    raise NotImplementedError("write your pallas kernel here")


# speedup vs baseline: 3.0826x; 3.0826x over previous
import functools

import jax
import jax.numpy as jnp
from jax import lax
from jax.experimental import pallas as pl
from jax.experimental.pallas import tpu as pltpu
from jax.experimental.pallas import tpu_sc as plsc

N_LAYERS = 10
C = 128
N_NODES = 10000
N_EDGES = 320000

NW = 32
CH = 128
NCH = 80
EPW = NCH * CH
E_PAD = NW * EPW
ROWS_PAD = 10240
RPT = ROWS_PAD // 16

BR = 2000
GRID = N_NODES // BR


def _sc_agg_body(m_hbm, src_hbm, dst_hbm, zeros_hbm, out_hbm,
                 srcb, dstb, rows, agg_s, sem):
    c = lax.axis_index("c")
    s = lax.axis_index("s")
    wid = s * 2 + c

    pltpu.sync_copy(zeros_hbm, agg_s.at[pl.ds(s * RPT, RPT)])
    plsc.subcore_barrier()

    pltpu.sync_copy(src_hbm.at[wid], srcb)
    pltpu.sync_copy(dst_hbm.at[wid], dstb)

    def chunk(j, carry):
        pltpu.async_copy(m_hbm.at[srcb.at[j]], rows, sem).wait()
        pltpu.sync_copy(rows, agg_s.at[dstb.at[j]], add=True)
        return carry

    lax.fori_loop(0, NCH, chunk, 0)

    plsc.subcore_barrier()
    pltpu.sync_copy(agg_s.at[pl.ds(s * RPT, RPT)],
                    out_hbm.at[c, pl.ds(s * RPT, RPT)])


@functools.cache
def _get_sc_agg():
    return functools.partial(
        pl.kernel,
        out_type=jax.ShapeDtypeStruct((2, ROWS_PAD, C), jnp.float32),
        mesh=plsc.VectorSubcoreMesh(core_axis_name="c", subcore_axis_name="s"),
        scratch_types=[
            pltpu.VMEM((NCH, CH), jnp.int32),
            pltpu.VMEM((NCH, CH), jnp.int32),
            pltpu.VMEM((CH, C), jnp.float32),
            pltpu.VMEM_SHARED((ROWS_PAD, C), jnp.float32),
            pltpu.SemaphoreType.DMA,
        ],
    )(_sc_agg_body)


def _mm_body(x_ref, w_ref, o_ref):
    o_ref[...] = jnp.dot(x_ref[...], w_ref[...],
                         preferred_element_type=jnp.float32)


_mm = pl.pallas_call(
    _mm_body,
    grid=(GRID,),
    in_specs=[pl.BlockSpec((BR, C), lambda i: (i, 0)),
              pl.BlockSpec((C, C), lambda i: (0, 0))],
    out_specs=pl.BlockSpec((BR, C), lambda i: (i, 0)),
    out_shape=jax.ShapeDtypeStruct((N_NODES, C), jnp.float32),
)


def _gru_math(p_ref, h_ref, wihT_ref, whhT_ref, bih_ref, bhh_ref):
    agg = p_ref[0] + p_ref[1]
    h = h_ref[...]
    gi = jnp.dot(agg, wihT_ref[...], preferred_element_type=jnp.float32)
    gi = gi + bih_ref[...]
    gh = jnp.dot(h, whhT_ref[...], preferred_element_type=jnp.float32)
    gh = gh + bhh_ref[...]
    r = jax.nn.sigmoid(gi[:, :C] + gh[:, :C])
    z = jax.nn.sigmoid(gi[:, C:2 * C] + gh[:, C:2 * C])
    n = jnp.tanh(gi[:, 2 * C:] + r * gh[:, 2 * C:])
    return (1.0 - z) * n + z * h


def _step_body(p_ref, h_ref, wihT_ref, whhT_ref, bih_ref, bhh_ref, wn_ref,
               h_out, m_out):
    hn = _gru_math(p_ref, h_ref, wihT_ref, whhT_ref, bih_ref, bhh_ref)
    h_out[...] = hn
    m_out[...] = jnp.dot(hn, wn_ref[...], preferred_element_type=jnp.float32)


def _final_body(p_ref, h_ref, wihT_ref, whhT_ref, bih_ref, bhh_ref, o_ref):
    hn = _gru_math(p_ref, h_ref, wihT_ref, whhT_ref, bih_ref, bhh_ref)
    o_ref[...] = jnp.maximum(hn, 0.0)


_COMMON_SPECS = [
    pl.BlockSpec((2, BR, C), lambda i: (0, i, 0)),
    pl.BlockSpec((BR, C), lambda i: (i, 0)),
    pl.BlockSpec((C, 3 * C), lambda i: (0, 0)),
    pl.BlockSpec((C, 3 * C), lambda i: (0, 0)),
    pl.BlockSpec((1, 3 * C), lambda i: (0, 0)),
    pl.BlockSpec((1, 3 * C), lambda i: (0, 0)),
]

_step = pl.pallas_call(
    _step_body,
    grid=(GRID,),
    in_specs=_COMMON_SPECS + [pl.BlockSpec((C, C), lambda i: (0, 0))],
    out_specs=[pl.BlockSpec((BR, C), lambda i: (i, 0)),
               pl.BlockSpec((BR, C), lambda i: (i, 0))],
    out_shape=[jax.ShapeDtypeStruct((N_NODES, C), jnp.float32),
               jax.ShapeDtypeStruct((N_NODES, C), jnp.float32)],
)

_final = pl.pallas_call(
    _final_body,
    grid=(GRID,),
    in_specs=_COMMON_SPECS,
    out_specs=pl.BlockSpec((BR, C), lambda i: (i, 0)),
    out_shape=jax.ShapeDtypeStruct((N_NODES, C), jnp.float32),
)


def kernel(x, edge_index, weight, w_ih, w_hh, b_ih, b_hh):
    src = edge_index[0].astype(jnp.int32)
    dst = edge_index[1].astype(jnp.int32)
    pad = E_PAD - N_EDGES
    src_p = jnp.concatenate([src, jnp.zeros((pad,), jnp.int32)])
    dst_p = jnp.concatenate([dst, jnp.full((pad,), N_NODES, jnp.int32)])
    src_p = src_p.reshape(NW, NCH, CH)
    dst_p = dst_p.reshape(NW, NCH, CH)
    zeros = jnp.zeros((RPT, C), jnp.float32)

    wihT = w_ih.T
    whhT = w_hh.T
    bih2 = b_ih.reshape(1, 3 * C)
    bhh2 = b_hh.reshape(1, 3 * C)

    sc_agg = _get_sc_agg()
    h = x
    m = _mm(h, weight[0])
    for i in range(N_LAYERS):
        p = sc_agg(m, src_p, dst_p, zeros)
        if i + 1 < N_LAYERS:
            h, m = _step(p, h, wihT, whhT, bih2, bhh2, weight[i + 1])
        else:
            h = _final(p, h, wihT, whhT, bih2, bhh2)
    return h
